# baseline (speedup 1.0000x reference)
import functools

import jax
import jax.numpy as jnp
from jax import lax
from jax.experimental import pallas as pl
from jax.experimental.pallas import tpu as pltpu
from jax.experimental.pallas import tpu_sc as plsc


_NC, _NS = 2, 16
_NW = _NC * _NS
_C = 8
_L = 16
_NSETS = 3


def kernel(x, pe_table):
    B, S, F = x.shape
    x2 = x.reshape(B * S, F)
    seq_per_w = S // _NW
    n_chunks = seq_per_w // _C
    KPF = F // _L
    KPF_BITS = KPF.bit_length() - 1
    mesh = plsc.VectorSubcoreMesh(core_axis_name="c", subcore_axis_name="s")

    scratch = (
        [pltpu.VMEM((_C, F), jnp.float32) for _ in range(_NSETS)]
        + [pltpu.VMEM((B * _C, F), jnp.float32) for _ in range(_NSETS)]
        + [pltpu.SemaphoreType.DMA for _ in range(2 * _NSETS)]
    )

    @functools.partial(
        pl.kernel,
        mesh=mesh,
        out_type=jax.ShapeDtypeStruct((B * S, F), jnp.float32),
        scratch_types=scratch,
    )
    def sc_add(x_hbm, pe_hbm, out_hbm, *refs):
        pebufs = refs[:_NSETS]
        xbufs = refs[_NSETS:2 * _NSETS]
        sems_in = refs[2 * _NSETS:2 * _NSETS + _NSETS]
        sems_out = refs[2 * _NSETS + _NSETS:]
        wid = lax.axis_index("s") * _NC + lax.axis_index("c")
        s0 = wid * seq_per_w

        def start_loads(c, s):
            row = s0 + c * _C
            ds = [pltpu.async_copy(pe_hbm.at[pl.ds(row, _C)], pebufs[s], sems_in[s])]
            for b in range(B):
                ds.append(
                    pltpu.async_copy(
                        x_hbm.at[pl.ds(b * S + row, _C)],
                        xbufs[s].at[pl.ds(b * _C, _C)],
                        sems_in[s],
                    )
                )
            return ds

        def start_stores(c, s):
            row = s0 + c * _C
            return [
                pltpu.async_copy(
                    xbufs[s].at[pl.ds(b * _C, _C)],
                    out_hbm.at[pl.ds(b * S + row, _C)],
                    sems_out[s],
                )
                for b in range(B)
            ]

        loads = {0: start_loads(0, 0)}
        stores = {}
        for c in range(n_chunks):
            s = c % _NSETS
            if c + 1 < n_chunks:
                sn = (c + 1) % _NSETS
                if c - 2 >= 0:
                    for d in stores.pop(c - 2):
                        d.wait()
                loads[c + 1] = start_loads(c + 1, sn)
            for d in loads.pop(c):
                d.wait()

            xbuf, pebuf = xbufs[s], pebufs[s]

            @plsc.parallel_loop(0, _C * KPF, unroll=16)
            def _(i):
                r = i >> KPF_BITS
                col = (i & (KPF - 1)) * _L
                v = pebuf[r, pl.ds(col, _L)]
                for b in range(B):
                    plsc.addupdate(xbuf.at[b * _C + r, pl.ds(col, _L)], v)

            stores[c] = start_stores(c, s)
        for c in sorted(stores):
            for d in stores[c]:
                d.wait()

    out = sc_add(x2, pe_table)
    return out.reshape(B, S, F)


# speedup vs baseline: 1.0386x; 1.0386x over previous
import functools

import jax
import jax.numpy as jnp
from jax import lax
from jax.experimental import pallas as pl
from jax.experimental.pallas import tpu as pltpu
from jax.experimental.pallas import tpu_sc as plsc


_NC, _NS = 2, 16
_NW = _NC * _NS
_C = 8
_L = 16
_NSETS = 3


def kernel(x, pe_table):
    B, S, F = x.shape
    x2 = x.reshape(B * S, F)
    seq_per_w = S // _NW
    n_chunks = seq_per_w // _C
    KPF = F // _L
    KPF_BITS = KPF.bit_length() - 1
    mesh = plsc.VectorSubcoreMesh(core_axis_name="c", subcore_axis_name="s")

    scratch = (
        [pltpu.VMEM((_C, F), jnp.float32) for _ in range(_NSETS)]
        + [pltpu.VMEM((B * _C, F), jnp.float32) for _ in range(_NSETS)]
        + [pltpu.SemaphoreType.DMA for _ in range(2 * _NSETS)]
    )

    @functools.partial(
        pl.kernel,
        mesh=mesh,
        out_type=jax.ShapeDtypeStruct((B * S, F), jnp.float32),
        scratch_types=scratch,
    )
    def sc_add(x_hbm, pe_hbm, out_hbm, *refs):
        pebufs = refs[:_NSETS]
        xbufs = refs[_NSETS:2 * _NSETS]
        sems_in = refs[2 * _NSETS:2 * _NSETS + _NSETS]
        sems_out = refs[2 * _NSETS + _NSETS:]
        wid = lax.axis_index("s") * _NC + lax.axis_index("c")
        s0 = wid * seq_per_w

        def start_loads(c, s):
            row = s0 + c * _C
            ds = [pltpu.async_copy(pe_hbm.at[pl.ds(row, _C)], pebufs[s], sems_in[s])]
            for b in range(B):
                ds.append(
                    pltpu.async_copy(
                        x_hbm.at[pl.ds(b * S + row, _C)],
                        xbufs[s].at[pl.ds(b * _C, _C)],
                        sems_in[s],
                    )
                )
            return ds

        def start_stores(c, s):
            row = s0 + c * _C
            return [
                pltpu.async_copy(
                    xbufs[s].at[pl.ds(b * _C, _C)],
                    out_hbm.at[pl.ds(b * S + row, _C)],
                    sems_out[s],
                )
                for b in range(B)
            ]

        loads = {0: start_loads(0, 0)}
        stores = {}
        for c in range(n_chunks):
            s = c % _NSETS
            if c + 1 < n_chunks:
                sn = (c + 1) % _NSETS
                if c - 2 >= 0:
                    for d in stores.pop(c - 2):
                        d.wait()
                loads[c + 1] = start_loads(c + 1, sn)
            for d in loads.pop(c):
                d.wait()

            xbuf, pebuf = xbufs[s], pebufs[s]

            @plsc.parallel_loop(0, _C * KPF, unroll=8)
            def _(i):
                r = i >> KPF_BITS
                col = (i & (KPF - 1)) * _L
                v = pebuf[r, pl.ds(col, _L)]
                for b in range(B):
                    plsc.addupdate(xbuf.at[b * _C + r, pl.ds(col, _L)], v)

            stores[c] = start_stores(c, s)
        for c in sorted(stores):
            for d in stores[c]:
                d.wait()

    out = sc_add(x2, pe_table)
    return out.reshape(B, S, F)
